# baseline (speedup 1.0000x reference)
import jax
import jax.numpy as jnp
from jax.experimental import pallas as pl
from jax.experimental.pallas import tpu as pltpu
from jax.experimental.pallas import tpu_sc as plsc

_B = 16384
_HIDDEN = 128
_BLOCK_B = 4096
_GATHER_W = 128


def _sc_gather_fn(table, idx2d):
    @pl.kernel(
        out_type=jax.ShapeDtypeStruct((_B, 2 * _HIDDEN), jnp.float32),
        mesh=plsc.VectorSubcoreMesh(core_axis_name="core",
                                    subcore_axis_name="subcore"),
    )
    def sc_kernel(table_hbm, i_hbm, o_hbm):
        def body(i_vmem, o_vmem):
            pltpu.sync_copy(table_hbm.at[i_vmem.at[0]], o_vmem)

        pltpu.emit_pipeline(
            body,
            grid=(_B // _GATHER_W,),
            in_specs=[pl.BlockSpec((1, _GATHER_W), index_map=lambda i: (0, i))],
            out_specs=[pl.BlockSpec((_GATHER_W, _HIDDEN),
                                    index_map=lambda i: (i, 1))],
            core_axis_name=("core", "subcore"),
            dimension_semantics=(pltpu.PARALLEL,),
        )(i_hbm, o_hbm)

    return sc_kernel(table, idx2d)


def _tc_mlp_kernel(buf_ref, mz_ref, w1_ref, b1_ref, w2_ref, b2_ref, out_ref):
    del buf_ref
    mz = mz_ref[:]
    h = jnp.maximum(mz[:, None] * w1_ref[0][None, :] + b1_ref[:][None, :], 0.0)
    out_ref[:, :] = jax.lax.dot_general(
        h, w2_ref[:],
        dimension_numbers=(((1,), (0,)), ((), ())),
        preferred_element_type=jnp.float32,
    ) + b2_ref[:][None, :]


def _tc_mlp_fn(buf, precursor_mz, W1, b1, W2, b2):
    grid = (_B // _BLOCK_B,)
    return pl.pallas_call(
        _tc_mlp_kernel,
        grid=grid,
        in_specs=[
            pl.BlockSpec(memory_space=pl.ANY),
            pl.BlockSpec((_BLOCK_B,), lambda i: (i,)),
            pl.BlockSpec(W1.shape, lambda i: (0, 0)),
            pl.BlockSpec(b1.shape, lambda i: (0,)),
            pl.BlockSpec(W2.shape, lambda i: (0, 0)),
            pl.BlockSpec(b2.shape, lambda i: (0,)),
        ],
        out_specs=pl.BlockSpec((_BLOCK_B, _HIDDEN), lambda i: (i, 0)),
        out_shape=jax.ShapeDtypeStruct((_B, 2 * _HIDDEN), jnp.float32),
        input_output_aliases={0: 0},
        compiler_params=pltpu.CompilerParams(
            dimension_semantics=("arbitrary",),
        ),
    )(buf, precursor_mz, W1, b1, W2, b2)


@jax.jit
def kernel(precursor_mz, charge, charge_table, W1, b1, W2, b2):
    idx2d = charge.astype(jnp.int32).reshape(1, _B)
    buf = _sc_gather_fn(charge_table, idx2d)
    out = _tc_mlp_fn(buf, precursor_mz, W1, b1, W2, b2)
    return out.reshape(_B, 2, _HIDDEN)


# speedup vs baseline: 1.0180x; 1.0180x over previous
import jax
import jax.numpy as jnp
from jax.experimental import pallas as pl
from jax.experimental.pallas import tpu as pltpu
from jax.experimental.pallas import tpu_sc as plsc

_B = 16384
_HIDDEN = 128
_BLOCK_B = 4096
_GATHER_W = 256


def _sc_gather_fn(table, idx2d):
    @pl.kernel(
        out_type=jax.ShapeDtypeStruct((_B, 2 * _HIDDEN), jnp.float32),
        mesh=plsc.VectorSubcoreMesh(core_axis_name="core",
                                    subcore_axis_name="subcore"),
    )
    def sc_kernel(table_hbm, i_hbm, o_hbm):
        def body(i_vmem, o_vmem):
            pltpu.sync_copy(table_hbm.at[i_vmem.at[0]], o_vmem)

        pltpu.emit_pipeline(
            body,
            grid=(_B // _GATHER_W,),
            in_specs=[pl.BlockSpec((1, _GATHER_W), index_map=lambda i: (0, i))],
            out_specs=[pl.BlockSpec((_GATHER_W, _HIDDEN),
                                    index_map=lambda i: (i, 1))],
            core_axis_name=("core", "subcore"),
            dimension_semantics=(pltpu.PARALLEL,),
        )(i_hbm, o_hbm)

    return sc_kernel(table, idx2d)


def _tc_mlp_kernel(buf_ref, mz_ref, w1_ref, b1_ref, w2_ref, b2_ref, out_ref):
    del buf_ref
    mz = mz_ref[:]
    h = jnp.maximum(mz[:, None] * w1_ref[0][None, :] + b1_ref[:][None, :], 0.0)
    out_ref[:, :] = jax.lax.dot_general(
        h, w2_ref[:],
        dimension_numbers=(((1,), (0,)), ((), ())),
        preferred_element_type=jnp.float32,
    ) + b2_ref[:][None, :]


def _tc_mlp_fn(buf, precursor_mz, W1, b1, W2, b2):
    grid = (_B // _BLOCK_B,)
    return pl.pallas_call(
        _tc_mlp_kernel,
        grid=grid,
        in_specs=[
            pl.BlockSpec(memory_space=pl.ANY),
            pl.BlockSpec((_BLOCK_B,), lambda i: (i,)),
            pl.BlockSpec(W1.shape, lambda i: (0, 0)),
            pl.BlockSpec(b1.shape, lambda i: (0,)),
            pl.BlockSpec(W2.shape, lambda i: (0, 0)),
            pl.BlockSpec(b2.shape, lambda i: (0,)),
        ],
        out_specs=pl.BlockSpec((_BLOCK_B, _HIDDEN), lambda i: (i, 0)),
        out_shape=jax.ShapeDtypeStruct((_B, 2 * _HIDDEN), jnp.float32),
        input_output_aliases={0: 0},
        compiler_params=pltpu.CompilerParams(
            dimension_semantics=("arbitrary",),
        ),
    )(buf, precursor_mz, W1, b1, W2, b2)


@jax.jit
def kernel(precursor_mz, charge, charge_table, W1, b1, W2, b2):
    idx2d = charge.astype(jnp.int32).reshape(1, _B)
    buf = _sc_gather_fn(charge_table, idx2d)
    out = _tc_mlp_fn(buf, precursor_mz, W1, b1, W2, b2)
    return out.reshape(_B, 2, _HIDDEN)


# speedup vs baseline: 4.0602x; 3.9885x over previous
import jax
import jax.numpy as jnp
from jax.experimental import pallas as pl
from jax.experimental.pallas import tpu as pltpu

_B = 16384
_HIDDEN = 128
_NUM_CHARGES = 11
_BLOCK_B = 4096


def _fused_kernel(mz_ref, charge_ref, table_ref, w1_ref, b1_ref, w2_ref,
                  b2_ref, out_ref):
    mz = mz_ref[:]
    h = jnp.maximum(mz[:, None] * w1_ref[0][None, :] + b1_ref[:][None, :], 0.0)
    emb0 = jax.lax.dot_general(
        h, w2_ref[:],
        dimension_numbers=(((1,), (0,)), ((), ())),
        preferred_element_type=jnp.float32,
    ) + b2_ref[:][None, :]

    charge = charge_ref[:]
    classes = jax.lax.broadcasted_iota(jnp.int32, (charge.shape[0],
                                                   _NUM_CHARGES), 1)
    onehot = (charge[:, None] == classes).astype(jnp.float32)
    emb1 = jax.lax.dot_general(
        onehot, table_ref[:],
        dimension_numbers=(((1,), (0,)), ((), ())),
        preferred_element_type=jnp.float32,
    )

    out_ref[:, :_HIDDEN] = emb0
    out_ref[:, _HIDDEN:] = emb1


@jax.jit
def kernel(precursor_mz, charge, charge_table, W1, b1, W2, b2):
    charge = charge.astype(jnp.int32)
    grid = (_B // _BLOCK_B,)
    out = pl.pallas_call(
        _fused_kernel,
        grid=grid,
        in_specs=[
            pl.BlockSpec((_BLOCK_B,), lambda i: (i,)),
            pl.BlockSpec((_BLOCK_B,), lambda i: (i,)),
            pl.BlockSpec(charge_table.shape, lambda i: (0, 0)),
            pl.BlockSpec(W1.shape, lambda i: (0, 0)),
            pl.BlockSpec(b1.shape, lambda i: (0,)),
            pl.BlockSpec(W2.shape, lambda i: (0, 0)),
            pl.BlockSpec(b2.shape, lambda i: (0,)),
        ],
        out_specs=pl.BlockSpec((_BLOCK_B, 2 * _HIDDEN), lambda i: (i, 0)),
        out_shape=jax.ShapeDtypeStruct((_B, 2 * _HIDDEN), jnp.float32),
        compiler_params=pltpu.CompilerParams(
            dimension_semantics=("arbitrary",),
        ),
    )(precursor_mz, charge, charge_table, W1, b1, W2, b2)
    return out.reshape(_B, 2, _HIDDEN)
